# Initial kernel scaffold; baseline (speedup 1.0000x reference)
import jax
import jax.numpy as jnp
from jax.experimental import pallas as pl


def kernel(theta, T, rand_prop_black, rand_acc_black, rand_prop_white, rand_acc_white, rand_pt):
    raise NotImplementedError("write your pallas kernel here")


# speedup vs baseline: 6.7538x; 6.7538x over previous
import functools

import jax
import jax.numpy as jnp
import numpy as np
from jax import lax
from jax.experimental import pallas as pl

_TWO_PI = 2.0 * np.pi


def _roll(x, s, axis):
    n = x.shape[axis]
    s = s % n
    if s == 0:
        return x
    if axis == 0:
        return jnp.concatenate([x[n - s:, :], x[: n - s, :]], axis=0)
    return jnp.concatenate([x[:, n - s:], x[:, : n - s]], axis=1)


def _mc_body(te_ref, to_ref, rpb_ref, rab_ref, rpw_ref, raw_ref, t_ref,
             rpt_ref, te_out_ref, to_out_ref, e_out_ref):
    L, H = te_ref.shape[2], te_ref.shape[3]
    meven = (lax.broadcasted_iota(jnp.int32, (L, H), 0) % 2) == 0

    results = []
    for j in (0, 1):
        t_j = t_ref[0, 0, 0, j]
        te = te_ref[j, 0]
        to = to_ref[j, 0]
        bang = jnp.where(meven, te, to) * _TWO_PI
        wang = jnp.where(meven, to, te) * _TWO_PI

        def sub_update(cur, other, prop_u, racc, flip):
            prop = prop_u * _TWO_PI
            n_up = _roll(other, 1, 0)
            n_dn = _roll(other, -1, 0)
            o_l = _roll(other, 1, 1)
            o_r = _roll(other, -1, 1)
            if flip:
                n_lf = jnp.where(meven, o_l, other)
                n_rt = jnp.where(meven, other, o_r)
            else:
                n_lf = jnp.where(meven, other, o_l)
                n_rt = jnp.where(meven, o_r, other)
            e_old = -(((jnp.cos(cur - n_up) + jnp.cos(cur - n_dn))
                       + jnp.cos(cur - n_lf)) + jnp.cos(cur - n_rt))
            e_new = -(((jnp.cos(prop - n_up) + jnp.cos(prop - n_dn))
                       + jnp.cos(prop - n_lf)) + jnp.cos(prop - n_rt))
            d_e = e_new - e_old
            acc = racc < jnp.exp(jnp.minimum(-d_e / t_j, 30.0))
            return jnp.where(acc, prop, cur)

        bnew = sub_update(bang, wang, rpb_ref[j, 0], rab_ref[j, 0], True)
        wnew = sub_update(wang, bnew, rpw_ref[j, 0], raw_ref[j, 0], False)

        w_dn = _roll(wnew, -1, 0)
        b_dn = _roll(bnew, -1, 0)
        w_rt = jnp.where(meven, wnew, _roll(wnew, -1, 1))
        b_rt = jnp.where(meven, _roll(bnew, -1, 1), bnew)
        e_img = -jnp.sum((jnp.cos(bnew - w_dn) + jnp.cos(bnew - w_rt))
                         + (jnp.cos(wnew - b_dn) + jnp.cos(wnew - b_rt)))
        results.append((bnew, wnew, e_img, t_j))

    (b0, w0, e0, t0), (b1, w1, e1, t1) = results
    dlt = (1.0 / t0 - 1.0 / t1) * (e0 - e1)
    accp = rpt_ref[0, 0, 0, 0] < jnp.exp(jnp.minimum(dlt, 30.0))
    finals = ((jnp.where(accp, b1, b0), jnp.where(accp, w1, w0)),
              (jnp.where(accp, b0, b1), jnp.where(accp, w0, w1)))
    e_out_ref[0, 0, 0] = jnp.full((8, 128), jnp.where(accp, e1, e0),
                                  dtype=jnp.float32)
    e_out_ref[0, 0, 1] = jnp.full((8, 128), jnp.where(accp, e0, e1),
                                  dtype=jnp.float32)
    for j, (bf, wf) in enumerate(finals):
        te_out_ref[j, 0] = jnp.where(meven, bf, wf)
        to_out_ref[j, 0] = jnp.where(meven, wf, bf)


@jax.jit
def kernel(theta, T, rand_prop_black, rand_acc_black, rand_prop_white,
           rand_acc_white, rand_pt):
    B, C, L, _ = theta.shape
    H = L // 2
    P = B // 2

    th_r = theta.reshape(B, C, L, H, 2)
    te = th_r[..., 0]
    to = th_r[..., 1]
    rpb = rand_prop_black.reshape(B, C, L, H)
    rab = rand_acc_black.reshape(B, C, L, H)
    rpw = rand_prop_white.reshape(B, C, L, H)
    raw = rand_acc_white.reshape(B, C, L, H)
    t4 = T.reshape(P, 1, 1, 2)
    rpt4 = rand_pt.reshape(P, C, 1, 1)

    big = pl.BlockSpec((2, 1, L, H), lambda p, c: (p, c, 0, 0))
    t_spec = pl.BlockSpec((1, 1, 1, 2), lambda p, c: (p, 0, 0, 0))
    rpt_spec = pl.BlockSpec((1, 1, 1, 1), lambda p, c: (p, c, 0, 0))
    e_spec = pl.BlockSpec((1, 1, 2, 8, 128), lambda p, c: (p, c, 0, 0, 0))

    te_out, to_out, e5 = pl.pallas_call(
        _mc_body,
        grid=(P, C),
        in_specs=[big, big, big, big, big, big, t_spec, rpt_spec],
        out_specs=[big, big, e_spec],
        out_shape=[
            jax.ShapeDtypeStruct((B, C, L, H), jnp.float32),
            jax.ShapeDtypeStruct((B, C, L, H), jnp.float32),
            jax.ShapeDtypeStruct((P, C, 2, 8, 128), jnp.float32),
        ],
    )(te, to, rpb, rab, rpw, raw, t4, rpt4)

    th_out = jnp.stack([te_out, to_out], axis=-1).reshape(B, C, L, L)
    e_out = e5[:, :, :, 0, 0].transpose(0, 2, 1).reshape(B, C)
    return th_out, e_out


# speedup vs baseline: 7.6375x; 1.1308x over previous
import functools

import jax
import jax.numpy as jnp
import numpy as np
from jax import lax
from jax.experimental import pallas as pl

_TWO_PI = 2.0 * np.pi


def _roll(x, s, axis):
    n = x.shape[axis]
    s = s % n
    if s == 0:
        return x
    if axis == 0:
        return jnp.concatenate([x[n - s:, :], x[: n - s, :]], axis=0)
    return jnp.concatenate([x[:, n - s:], x[:, : n - s]], axis=1)


def _sincos_turn(u):
    t4 = u * 4.0
    qf = jnp.floor(t4 + 0.5)
    r = u - qf * 0.25
    qi = qf.astype(jnp.int32)
    b0 = (qi & 1) != 0
    b1 = (qi & 2) != 0
    s2 = r * r
    c = 1.0 + s2 * (-19.739208802178716 + s2 * (64.93939402266829
        + s2 * (-85.45681720669371 + s2 * 60.24464137187666)))
    s = r * (6.283185307179586 + s2 * (-41.34170224039975
        + s2 * (81.60524927607504 + s2 * (-76.70585975306136
        + s2 * 42.05869394489765))))
    x = jnp.where(b0, s, c)
    y = jnp.where(b0, c, s)
    cos_v = jnp.where(b0 ^ b1, -x, x)
    sin_v = jnp.where(b1, -y, y)
    return sin_v, cos_v


def _mc_body(te_ref, to_ref, rpb_ref, rab_ref, rpw_ref, raw_ref, t_ref,
             rpt_ref, te_out_ref, to_out_ref, e_out_ref):
    L, H = te_ref.shape[2], te_ref.shape[3]
    meven = (lax.broadcasted_iota(jnp.int32, (L, H), 0) % 2) == 0

    results = []
    for j in (0, 1):
        t_j = t_ref[0, 0, 0, j]
        te = te_ref[j, 0]
        to = to_ref[j, 0]
        b_u = jnp.where(meven, te, to)
        w_u = jnp.where(meven, to, te)

        def sub_update(cur_u, other_u, prop_u, racc, flip):
            cur = cur_u * _TWO_PI
            prop = prop_u * _TWO_PI
            other = other_u * _TWO_PI
            n_up = _roll(other, 1, 0)
            n_dn = _roll(other, -1, 0)
            o_l = _roll(other, 1, 1)
            o_r = _roll(other, -1, 1)
            if flip:
                n_lf = jnp.where(meven, o_l, other)
                n_rt = jnp.where(meven, other, o_r)
            else:
                n_lf = jnp.where(meven, other, o_l)
                n_rt = jnp.where(meven, o_r, other)
            e_old = -(((jnp.cos(cur - n_up) + jnp.cos(cur - n_dn))
                       + jnp.cos(cur - n_lf)) + jnp.cos(cur - n_rt))
            e_new = -(((jnp.cos(prop - n_up) + jnp.cos(prop - n_dn))
                       + jnp.cos(prop - n_lf)) + jnp.cos(prop - n_rt))
            d_e = e_new - e_old
            acc = racc < jnp.exp(jnp.minimum(-d_e / t_j, 30.0))
            return jnp.where(acc, prop_u, cur_u)

        bn_u = sub_update(b_u, w_u, rpb_ref[j, 0], rab_ref[j, 0], True)
        wn_u = sub_update(w_u, bn_u, rpw_ref[j, 0], raw_ref[j, 0], False)
        bnew = bn_u * _TWO_PI
        wnew = wn_u * _TWO_PI

        sb, cb = _sincos_turn(bn_u)
        sw, cw = _sincos_turn(wn_u)
        bond = (cb * (_roll(cw, -1, 0) + jnp.where(meven, cw, _roll(cw, -1, 1)))
                + sb * (_roll(sw, -1, 0) + jnp.where(meven, sw, _roll(sw, -1, 1)))
                + cw * (_roll(cb, -1, 0) + jnp.where(meven, _roll(cb, -1, 1), cb))
                + sw * (_roll(sb, -1, 0) + jnp.where(meven, _roll(sb, -1, 1), sb)))
        e_img = -jnp.sum(bond)
        results.append((bnew, wnew, e_img, t_j))

    (b0, w0, e0, t0), (b1, w1, e1, t1) = results
    dlt = (1.0 / t0 - 1.0 / t1) * (e0 - e1)
    accp = rpt_ref[0, 0, 0, 0] < jnp.exp(jnp.minimum(dlt, 30.0))
    finals = ((jnp.where(accp, b1, b0), jnp.where(accp, w1, w0)),
              (jnp.where(accp, b0, b1), jnp.where(accp, w0, w1)))
    e_out_ref[0, 0, 0] = jnp.full((8, 128), jnp.where(accp, e1, e0),
                                  dtype=jnp.float32)
    e_out_ref[0, 0, 1] = jnp.full((8, 128), jnp.where(accp, e0, e1),
                                  dtype=jnp.float32)
    for j, (bf, wf) in enumerate(finals):
        te_out_ref[j, 0] = jnp.where(meven, bf, wf)
        to_out_ref[j, 0] = jnp.where(meven, wf, bf)


@jax.jit
def kernel(theta, T, rand_prop_black, rand_acc_black, rand_prop_white,
           rand_acc_white, rand_pt):
    B, C, L, _ = theta.shape
    H = L // 2
    P = B // 2

    th_r = theta.reshape(B, C, L, H, 2)
    te = th_r[..., 0]
    to = th_r[..., 1]
    rpb = rand_prop_black.reshape(B, C, L, H)
    rab = rand_acc_black.reshape(B, C, L, H)
    rpw = rand_prop_white.reshape(B, C, L, H)
    raw = rand_acc_white.reshape(B, C, L, H)
    t4 = T.reshape(P, 1, 1, 2)
    rpt4 = rand_pt.reshape(P, C, 1, 1)

    big = pl.BlockSpec((2, 1, L, H), lambda p, c: (p, c, 0, 0))
    t_spec = pl.BlockSpec((1, 1, 1, 2), lambda p, c: (p, 0, 0, 0))
    rpt_spec = pl.BlockSpec((1, 1, 1, 1), lambda p, c: (p, c, 0, 0))
    e_spec = pl.BlockSpec((1, 1, 2, 8, 128), lambda p, c: (p, c, 0, 0, 0))

    te_out, to_out, e5 = pl.pallas_call(
        _mc_body,
        grid=(P, C),
        in_specs=[big, big, big, big, big, big, t_spec, rpt_spec],
        out_specs=[big, big, e_spec],
        out_shape=[
            jax.ShapeDtypeStruct((B, C, L, H), jnp.float32),
            jax.ShapeDtypeStruct((B, C, L, H), jnp.float32),
            jax.ShapeDtypeStruct((P, C, 2, 8, 128), jnp.float32),
        ],
    )(te, to, rpb, rab, rpw, raw, t4, rpt4)

    th_out = jnp.stack([te_out, to_out], axis=-1).reshape(B, C, L, L)
    e_out = e5[:, :, :, 0, 0].transpose(0, 2, 1).reshape(B, C)
    return th_out, e_out
